# baseline (speedup 1.0000x reference)
import functools

import jax
import jax.numpy as jnp
from jax import lax
from jax.experimental import pallas as pl
from jax.experimental.pallas import tpu as pltpu
from jax.experimental.pallas import tpu_sc as plsc

H = 128
_BR = 512

try:
    _sci = plsc.get_sparse_core_info()
    _NC, _NS = _sci.num_cores, _sci.num_subcores
except Exception:
    _NC, _NS = 2, 16
_NW = _NC * _NS
_CH = 128
_HC = H // _NC


def _cdiv(a, b):
    return (a + b - 1) // b


def _ln(y, g, be):
    mu = jnp.mean(y, axis=-1, keepdims=True)
    var = jnp.mean((y - mu) ** 2, axis=-1, keepdims=True)
    return (y - mu) * lax.rsqrt(var + 1e-5) * g + be


def _dot(a, b):
    return jnp.dot(a, b, preferred_element_type=jnp.float32)


def _mlp_body(x_ref, w1_ref, b1_ref, w2_ref, b2_ref, w3_ref, b3_ref,
              g_ref, be_ref, o_ref, *, ln):
    h = jnp.maximum(_dot(x_ref[...], w1_ref[...]) + b1_ref[...], 0.0)
    h = jnp.maximum(_dot(h, w2_ref[...]) + b2_ref[...], 0.0)
    y = _dot(h, w3_ref[...]) + b3_ref[...]
    if ln:
        y = _ln(y, g_ref[...], be_ref[...])
    o_ref[...] = y


def _full(shape):
    return pl.BlockSpec(shape, lambda i: (0,) * len(shape))


def _rows(br, cols):
    return pl.BlockSpec((br, cols), lambda i: (i, 0))


def _mlp_full(x, p, ln, d_out=H):
    n, d_in = x.shape
    dp = max(8, _cdiv(d_in, 8) * 8)
    if d_in != dp:
        x = jnp.pad(x, ((0, 0), (0, dp - d_in)))
    (w1, b1), (w2, b2), (w3, b3) = p["lin"]
    w1 = jnp.pad(w1, ((0, dp - d_in), (0, 0)))
    if w3.shape[1] != H:
        w3 = jnp.pad(w3, ((0, 0), (0, H - w3.shape[1])))
        b3 = jnp.pad(b3, ((0, H - b3.shape[0]),))
    if p["ln"] is not None:
        g, be = p["ln"]
    else:
        g = be = jnp.zeros((H,), jnp.float32)
    br = min(_BR, _cdiv(n, 8) * 8)
    out = pl.pallas_call(
        functools.partial(_mlp_body, ln=ln),
        grid=(_cdiv(n, br),),
        in_specs=[_rows(br, dp), _full((dp, H)), _full((1, H)),
                  _full((H, H)), _full((1, H)), _full((H, H)), _full((1, H)),
                  _full((1, H)), _full((1, H))],
        out_specs=_rows(br, H),
        out_shape=jax.ShapeDtypeStruct((n, H), jnp.float32),
    )(x, w1, b1.reshape(1, H), w2, b2.reshape(1, H), w3, b3.reshape(1, H),
      g.reshape(1, H), be.reshape(1, H))
    return out[:, :d_out] if d_out != H else out


def _matmul_body(x_ref, w_ref, b_ref, o_ref):
    o_ref[...] = _dot(x_ref[...], w_ref[...]) + b_ref[...]


def _multi_matmul_body(x_ref, w_ref, *o_refs):
    x = x_ref[...]
    for r, o_ref in enumerate(o_refs):
        o_ref[...] = _dot(x, w_ref[r])


def _multi_matmul(x, ws):
    n = x.shape[0]
    k = len(ws)
    br = min(_BR, _cdiv(n, 8) * 8)
    rows = _rows(br, H)
    return pl.pallas_call(
        _multi_matmul_body,
        grid=(_cdiv(n, br),),
        in_specs=[rows, _full((k, H, H))],
        out_specs=(rows,) * k,
        out_shape=(jax.ShapeDtypeStruct((n, H), jnp.float32),) * k,
    )(x, jnp.stack(ws))


def _matmul(x, w, b=None):
    n = x.shape[0]
    if b is None:
        b = jnp.zeros((H,), jnp.float32)
    br = min(_BR, _cdiv(n, 8) * 8)
    return pl.pallas_call(
        _matmul_body,
        grid=(_cdiv(n, br),),
        in_specs=[_rows(br, H), _full((H, H)), _full((1, H))],
        out_specs=_rows(br, H),
        out_shape=jax.ShapeDtypeStruct((n, H), jnp.float32),
    )(x, w, b.reshape(1, H))


def _tail_body(a_ref, b_ref, e_ref, w1e_ref, b1_ref, w2_ref, b2_ref,
               w3_ref, b3_ref, g_ref, be_ref, *o_refs,
               ln, n_valid, br, split):
    c = _dot(e_ref[...], w1e_ref[...]) + b1_ref[...]
    h = jnp.maximum(a_ref[...] + b_ref[...] + c, 0.0)
    h = jnp.maximum(_dot(h, w2_ref[...]) + b2_ref[...], 0.0)
    y = _dot(h, w3_ref[...]) + b3_ref[...]
    if ln:
        y = _ln(y, g_ref[...], be_ref[...])
    if n_valid is not None:
        row = (pl.program_id(0) * br
               + lax.broadcasted_iota(jnp.int32, y.shape, 0))
        y = jnp.where(row < n_valid, y, 0.0)
    if split:
        o_refs[0][...] = y[:, :_HC]
        o_refs[1][...] = y[:, _HC:]
    else:
        o_refs[0][...] = y


def _tail(ag, bg, e_enc, p, ln, d_out=H, n_valid=None, split=False):
    n = ag.shape[0]
    (w1, b1), (w2, b2), (w3, b3) = p["lin"]
    w1e = w1[2 * H:]
    if w3.shape[1] != H:
        w3 = jnp.pad(w3, ((0, 0), (0, H - w3.shape[1])))
        b3 = jnp.pad(b3, ((0, H - b3.shape[0]),))
    if p["ln"] is not None:
        g, be = p["ln"]
    else:
        g = be = jnp.zeros((H,), jnp.float32)
    br = min(_BR, _cdiv(n, 8) * 8)
    if split:
        out_specs = (_rows(br, _HC), _rows(br, _HC))
        out_shape = (jax.ShapeDtypeStruct((n, _HC), jnp.float32),
                     jax.ShapeDtypeStruct((n, _HC), jnp.float32))
    else:
        out_specs = _rows(br, H)
        out_shape = jax.ShapeDtypeStruct((n, H), jnp.float32)
    out = pl.pallas_call(
        functools.partial(_tail_body, ln=ln, n_valid=n_valid, br=br,
                          split=split),
        grid=(_cdiv(n, br),),
        in_specs=[_rows(br, H), _rows(br, H), _rows(br, H),
                  _full((H, H)), _full((1, H)),
                  _full((H, H)), _full((1, H)), _full((H, H)), _full((1, H)),
                  _full((1, H)), _full((1, H))],
        out_specs=out_specs,
        out_shape=out_shape,
    )(ag, bg, e_enc, w1e, b1.reshape(1, H), w2, b2.reshape(1, H),
      w3, b3.reshape(1, H), g.reshape(1, H), be.reshape(1, H))
    if split:
        return out
    return out[:, :d_out] if d_out != H else out


def _node_body(*refs, nrel):
    x_ref = refs[0]
    a_refs = refs[1:1 + 2 * nrel]
    w1x, w1a, b1, w2, b2, w3, b3, g, be = refs[1 + 2 * nrel:10 + 2 * nrel]
    o_ref = refs[10 + 2 * nrel]
    x = x_ref[...]
    acc = x
    for r in range(nrel):
        a = jnp.concatenate([a_refs[2 * r][...], a_refs[2 * r + 1][...]],
                            axis=-1)
        h = jnp.maximum(_dot(x, w1x[r]) + _dot(a, w1a[r]) + b1[r], 0.0)
        h = jnp.maximum(_dot(h, w2[r]) + b2[r], 0.0)
        acc = acc + _ln(_dot(h, w3[r]) + b3[r], g[r], be[r])
    o_ref[...] = acc


def _node_update(x, aggrs, ps):
    n = x.shape[0]
    nrel = len(aggrs)
    w1x = jnp.stack([p["lin"][0][0][:H] for p in ps])
    w1a = jnp.stack([p["lin"][0][0][H:] for p in ps])
    b1 = jnp.stack([p["lin"][0][1].reshape(1, H) for p in ps])
    w2 = jnp.stack([p["lin"][1][0] for p in ps])
    b2 = jnp.stack([p["lin"][1][1].reshape(1, H) for p in ps])
    w3 = jnp.stack([p["lin"][2][0] for p in ps])
    b3 = jnp.stack([p["lin"][2][1].reshape(1, H) for p in ps])
    g = jnp.stack([p["ln"][0].reshape(1, H) for p in ps])
    be = jnp.stack([p["ln"][1].reshape(1, H) for p in ps])
    br = min(_BR, _cdiv(n, 8) * 8)
    rows = _rows(br, H)
    half = _rows(br, _HC)
    flat_aggrs = [h for pair in aggrs for h in pair]
    return pl.pallas_call(
        functools.partial(_node_body, nrel=nrel),
        grid=(_cdiv(n, br),),
        in_specs=[rows] + [half] * (2 * nrel)
        + [_full((nrel, H, H)), _full((nrel, H, H)), _full((nrel, 1, H)),
           _full((nrel, H, H)), _full((nrel, 1, H)),
           _full((nrel, H, H)), _full((nrel, 1, H)),
           _full((nrel, 1, H)), _full((nrel, 1, H))],
        out_specs=rows,
        out_shape=jax.ShapeDtypeStruct((n, H), jnp.float32),
    )(x, *flat_aggrs, w1x, w1a, b1, w2, b2, w3, b3, g, be)


def _pad_edges(e):
    m = _NW * _CH
    return _cdiv(e, m) * m


def _prep_idx(idx, e_pad, groups):
    idx = idx.astype(jnp.int32)
    return jnp.pad(idx, (0, e_pad - idx.shape[0])).reshape(
        groups, e_pad // (groups * _CH), _CH)


@functools.lru_cache(maxsize=None)
def _make_dual_gather(n1, n2, e_pad):
    nck = e_pad // (_NW * _CH)
    mesh = plsc.VectorSubcoreMesh(core_axis_name="c", subcore_axis_name="s")

    @functools.partial(
        pl.kernel, mesh=mesh,
        out_type=(jax.ShapeDtypeStruct((e_pad, H), jnp.float32),
                  jax.ShapeDtypeStruct((e_pad, H), jnp.float32)),
        scratch_types=[pltpu.VMEM((nck, _CH), jnp.int32),
                       pltpu.VMEM((nck, _CH), jnp.int32),
                       pltpu.VMEM((_CH, H), jnp.float32),
                       pltpu.VMEM((_CH, H), jnp.float32),
                       pltpu.SemaphoreType.DMA,
                       pltpu.SemaphoreType.DMA],
    )
    def k(t1, i1, t2, i2, o1, o2, iv1, iv2, b1, b2, s1, s2):
        wid = lax.axis_index("s") * _NC + lax.axis_index("c")
        pltpu.sync_copy(i1.at[wid], iv1)
        pltpu.sync_copy(i2.at[wid], iv2)

        def chunk(j, carry):
            row0 = (wid * nck + j) * _CH
            cp1 = pltpu.async_copy(t1.at[iv1.at[j]], b1, s1)
            cp2 = pltpu.async_copy(t2.at[iv2.at[j]], b2, s2)
            cp1.wait()
            pltpu.sync_copy(b1, o1.at[pl.ds(row0, _CH)])
            cp2.wait()
            pltpu.sync_copy(b2, o2.at[pl.ds(row0, _CH)])
            return carry

        lax.fori_loop(0, nck, chunk, 0)

    return k


@functools.lru_cache(maxsize=None)
def _make_dual_gather_pipe(n1, n2, e_pad):
    nck = e_pad // (_NW * _CH)
    mesh = plsc.VectorSubcoreMesh(core_axis_name="c", subcore_axis_name="s")

    @functools.partial(
        pl.kernel, mesh=mesh,
        out_type=(jax.ShapeDtypeStruct((e_pad, H), jnp.float32),
                  jax.ShapeDtypeStruct((e_pad, H), jnp.float32)),
        scratch_types=[pltpu.VMEM((nck, _CH), jnp.int32),
                       pltpu.VMEM((nck, _CH), jnp.int32),
                       pltpu.VMEM((_CH, H), jnp.float32),
                       pltpu.VMEM((_CH, H), jnp.float32),
                       pltpu.VMEM((_CH, H), jnp.float32),
                       pltpu.VMEM((_CH, H), jnp.float32),
                       pltpu.SemaphoreType.DMA, pltpu.SemaphoreType.DMA,
                       pltpu.SemaphoreType.DMA, pltpu.SemaphoreType.DMA],
    )
    def k(t1, i1, t2, i2, o1, o2, iv1, iv2, a0, a1, b0, b1,
          sa0, sa1, sb0, sb1):
        wid = lax.axis_index("s") * _NC + lax.axis_index("c")
        pltpu.sync_copy(i1.at[wid], iv1)
        pltpu.sync_copy(i2.at[wid], iv2)
        base = wid * nck

        def issue(j, ba, bb, sa, sb):
            pltpu.async_copy(t1.at[iv1.at[j]], ba, sa)
            pltpu.async_copy(t2.at[iv2.at[j]], bb, sb)

        def wait_a(buf, sem):
            pltpu.make_async_copy(t1.at[iv1.at[0]], buf, sem).wait()

        def wait_b(buf, sem):
            pltpu.make_async_copy(t2.at[iv2.at[0]], buf, sem).wait()

        issue(0, a0, b0, sa0, sb0)
        if nck > 1:
            issue(1, a1, b1, sa1, sb1)

        def half(k0, ba, bb, sa, sb):
            wait_a(ba, sa)
            pltpu.sync_copy(ba, o1.at[pl.ds((base + k0) * _CH, _CH)])

            @pl.when(k0 + 2 < nck)
            def _():
                pltpu.async_copy(t1.at[iv1.at[k0 + 2]], ba, sa)

            wait_b(bb, sb)
            pltpu.sync_copy(bb, o2.at[pl.ds((base + k0) * _CH, _CH)])

            @pl.when(k0 + 2 < nck)
            def _():
                pltpu.async_copy(t2.at[iv2.at[k0 + 2]], bb, sb)

        def body(jj, carry):
            half(2 * jj, a0, b0, sa0, sb0)
            half(2 * jj + 1, a1, b1, sa1, sb1)
            return carry

        lax.fori_loop(0, nck // 2, body, 0)
        if nck % 2 == 1:
            j = nck - 1
            wait_a(a0, sa0)
            pltpu.sync_copy(a0, o1.at[pl.ds((base + j) * _CH, _CH)])
            wait_b(b0, sb0)
            pltpu.sync_copy(b0, o2.at[pl.ds((base + j) * _CH, _CH)])

    return k


_PIPELINE = False
_PROBE_MIN = True


def _dual_gather(tab1, idx1_3d, tab2, idx2_3d):
    e_pad = idx1_3d.shape[0] * idx1_3d.shape[1] * _CH
    mk = _make_dual_gather_pipe if _PIPELINE else _make_dual_gather
    return mk(tab1.shape[0], tab2.shape[0], e_pad)(
        tab1, idx1_3d, tab2, idx2_3d)


@functools.lru_cache(maxsize=None)
def _make_scatter(e_pad, n_pad):
    nck = e_pad // (_NS * _CH)
    rows_pt = n_pad // _NS
    mesh = plsc.VectorSubcoreMesh(core_axis_name="c", subcore_axis_name="s")

    @functools.partial(
        pl.kernel, mesh=mesh,
        out_type=(jax.ShapeDtypeStruct((n_pad, _HC), jnp.float32),
                  jax.ShapeDtypeStruct((n_pad, _HC), jnp.float32)),
        scratch_types=[pltpu.VMEM((nck, _CH), jnp.int32),
                       pltpu.VMEM((_CH, _HC), jnp.float32),
                       pltpu.VMEM_SHARED((n_pad, _HC), jnp.float32)],
    )
    def k(msg_lo, msg_hi, idx, zeros, out_lo, out_hi, iv, mbuf, acc):
        c = lax.axis_index("c")
        s = lax.axis_index("s")
        r0 = s * rows_pt
        if _PROBE_MIN:
            pltpu.sync_copy(zeros.at[pl.ds(0, min(_CH, rows_pt))],
                            mbuf.at[pl.ds(0, min(_CH, rows_pt))])
            for r in range(0, rows_pt, _CH):
                sz = min(_CH, rows_pt - r)
                pltpu.sync_copy(mbuf.at[pl.ds(0, sz)],
                                acc.at[pl.ds(r0 + r, sz)])
            plsc.subcore_barrier()
            for r in range(0, rows_pt, _CH):
                sz = min(_CH, rows_pt - r)
                pltpu.sync_copy(acc.at[pl.ds(r0 + r, sz)],
                                mbuf.at[pl.ds(0, sz)])

                @pl.when(c == 0)
                def _():
                    pltpu.sync_copy(mbuf.at[pl.ds(0, sz)],
                                    out_lo.at[pl.ds(r0 + r, sz)])

                @pl.when(c == 1)
                def _():
                    pltpu.sync_copy(mbuf.at[pl.ds(0, sz)],
                                    out_hi.at[pl.ds(r0 + r, sz)])
            return
        pltpu.sync_copy(zeros.at[pl.ds(0, min(_CH, rows_pt))],
                        mbuf.at[pl.ds(0, min(_CH, rows_pt))])
        for r in range(0, rows_pt, _CH):
            sz = min(_CH, rows_pt - r)
            pltpu.sync_copy(mbuf.at[pl.ds(0, sz)],
                            acc.at[pl.ds(r0 + r, sz)])
        pltpu.sync_copy(idx.at[s], iv)
        plsc.subcore_barrier()

        def chunk(j, carry):
            row0 = (s * nck + j) * _CH
            rows = pl.ds(row0, _CH)

            @pl.when(c == 0)
            def _():
                pltpu.sync_copy(msg_lo.at[rows], mbuf)

            @pl.when(c == 1)
            def _():
                pltpu.sync_copy(msg_hi.at[rows], mbuf)

            pltpu.sync_copy(mbuf, acc.at[iv.at[j]], add=True)
            return carry

        lax.fori_loop(0, nck, chunk, 0)
        plsc.subcore_barrier()
        for r in range(0, rows_pt, _CH):
            sz = min(_CH, rows_pt - r)
            pltpu.sync_copy(acc.at[pl.ds(r0 + r, sz)],
                            mbuf.at[pl.ds(0, sz)])

            @pl.when(c == 0)
            def _():
                pltpu.sync_copy(mbuf.at[pl.ds(0, sz)],
                                out_lo.at[pl.ds(r0 + r, sz)])

            @pl.when(c == 1)
            def _():
                pltpu.sync_copy(mbuf.at[pl.ds(0, sz)],
                                out_hi.at[pl.ds(r0 + r, sz)])

    return k


@functools.lru_cache(maxsize=None)
def _make_scatter_pipe(e_pad, n_pad):
    nck = e_pad // (_NS * _CH)
    rows_pt = n_pad // _NS
    mesh = plsc.VectorSubcoreMesh(core_axis_name="c", subcore_axis_name="s")

    @functools.partial(
        pl.kernel, mesh=mesh,
        out_type=(jax.ShapeDtypeStruct((n_pad, _HC), jnp.float32),
                  jax.ShapeDtypeStruct((n_pad, _HC), jnp.float32)),
        scratch_types=[pltpu.VMEM((nck, _CH), jnp.int32),
                       pltpu.VMEM((_CH, _HC), jnp.float32),
                       pltpu.VMEM((_CH, _HC), jnp.float32),
                       pltpu.VMEM_SHARED((n_pad, _HC), jnp.float32),
                       pltpu.SemaphoreType.DMA, pltpu.SemaphoreType.DMA],
    )
    def k(msg_lo, msg_hi, idx, zeros, out_lo, out_hi, iv, m0, m1, acc,
          sm0, sm1):
        c = lax.axis_index("c")
        s = lax.axis_index("s")
        r0 = s * rows_pt
        pltpu.sync_copy(zeros.at[pl.ds(r0, rows_pt)],
                        acc.at[pl.ds(r0, rows_pt)])
        pltpu.sync_copy(idx.at[s], iv)
        plsc.subcore_barrier()

        def load(j, buf, sem):
            rows = pl.ds((s * nck + j) * _CH, _CH)

            @pl.when(c == 0)
            def _():
                pltpu.async_copy(msg_lo.at[rows], buf, sem)

            @pl.when(c == 1)
            def _():
                pltpu.async_copy(msg_hi.at[rows], buf, sem)

        def wait_m(buf, sem):
            pltpu.make_async_copy(msg_lo.at[pl.ds(0, _CH)], buf, sem).wait()

        load(0, m0, sm0)
        load(1, m1, sm1)

        def half(k0, buf, sem):
            wait_m(buf, sem)
            pltpu.sync_copy(buf, acc.at[iv.at[pl.ds(k0, 1)]], add=True)

            @pl.when(k0 + 2 < nck)
            def _():
                load_j = k0 + 2
                rows = pl.ds((s * nck + load_j) * _CH, _CH)

                @pl.when(c == 0)
                def _():
                    pltpu.async_copy(msg_lo.at[rows], buf, sem)

                @pl.when(c == 1)
                def _():
                    pltpu.async_copy(msg_hi.at[rows], buf, sem)

        def body(jj, carry):
            half(2 * jj, m0, sm0)
            half(2 * jj + 1, m1, sm1)
            return carry

        lax.fori_loop(0, nck // 2, body, 0)
        plsc.subcore_barrier()
        rows = pl.ds(r0, rows_pt)

        @pl.when(c == 0)
        def _():
            pltpu.sync_copy(acc.at[rows], out_lo.at[rows])

        @pl.when(c == 1)
        def _():
            pltpu.sync_copy(acc.at[rows], out_hi.at[rows])

    return k


def _segment_sum(msg_halves, dst_3d, n):
    msg_lo, msg_hi = msg_halves
    n_pad = _cdiv(n, _NS * 8) * (_NS * 8)
    dst = dst_3d.reshape(-1)
    lo = jax.ops.segment_sum(msg_lo, dst, num_segments=n_pad)
    hi = jax.ops.segment_sum(msg_hi, dst, num_segments=n_pad)
    return lo, hi


def _edge_phase(p_in, a, b, idxs, e_enc, e_true, n_dst):
    src_g, dst_g, dst_s = idxs
    em = p_in["edge_mlp"]
    ag, bg = _dual_gather(a, dst_g, b, src_g)
    msg = _tail(ag, bg, e_enc, em, ln=True, n_valid=e_true, split=True)
    return _segment_sum(msg, dst_s, n_dst)


def _contact_dec(a, b, idxs, e_enc, e_true, p):
    src_g, dst_g, _ = idxs
    ag, bg = _dual_gather(a, src_g, b, dst_g)
    return _tail(ag, bg, e_enc, p, ln=False, d_out=1, n_valid=e_true)[:e_true]


def kernel(world_x, object_x, floor_x, w2o_edge_index, w2o_edge_attr,
           w2f_edge_index, w2f_edge_attr, oo_edge_index, oo_edge_attr,
           fo_edge_index, fo_edge_attr, params):
    e_w2o_n, e_w2f_n = w2o_edge_attr.shape[0], w2f_edge_attr.shape[0]
    e_oo_n, e_fo_n = oo_edge_attr.shape[0], fo_edge_attr.shape[0]
    p_w2o, p_w2f = _pad_edges(e_w2o_n), _pad_edges(e_w2f_n)
    p_oo, p_fo = _pad_edges(e_oo_n), _pad_edges(e_fo_n)

    def pad_attr(x, e_pad):
        return jnp.pad(x, ((0, e_pad - x.shape[0]), (0, 0)))

    def prep(ei, e_pad):
        return (_prep_idx(ei[0], e_pad, _NW),
                _prep_idx(ei[1], e_pad, _NW),
                _prep_idx(ei[1], e_pad, _NS))

    w2o_idx = prep(w2o_edge_index, p_w2o)
    w2f_idx = prep(w2f_edge_index, p_w2f)
    oo_idx = prep(oo_edge_index, p_oo)
    fo_idx = prep(fo_edge_index, p_fo)

    xw = _mlp_full(world_x, params["node_enc_world"], ln=True)
    xo = _mlp_full(object_x, params["node_enc_object"], ln=True)
    xf = _mlp_full(floor_x, params["node_enc_floor"], ln=True)
    e_w2o = _mlp_full(pad_attr(w2o_edge_attr, p_w2o),
                      params["edge_enc_w2o"], ln=True)
    e_w2f = _mlp_full(pad_attr(w2f_edge_attr, p_w2f),
                      params["edge_enc_w2f"], ln=True)
    e_oo = _mlp_full(pad_attr(oo_edge_attr, p_oo),
                     params["edge_enc_contact"], ln=True)
    e_fo = _mlp_full(pad_attr(fo_edge_attr, p_fo),
                     params["edge_enc_contact"], ln=True)

    n_obj = xo.shape[0]
    n_floor = xf.shape[0]
    for p in params["processor"]:
        w1_w2o = p["w2o"]["edge_mlp"]["lin"][0][0]
        w1_oo = p["oo"]["edge_mlp"]["lin"][0][0]
        w1_fo = p["fo"]["edge_mlp"]["lin"][0][0]
        w1_w2f = p["w2f"]["edge_mlp"]["lin"][0][0]
        a_w2o, a_oo, b_oo, a_fo = _multi_matmul(
            xo, [w1_w2o[:H], w1_oo[:H], w1_oo[H:2 * H], w1_fo[:H]])
        b_w2o, b_w2f = _multi_matmul(
            xw, [w1_w2o[H:2 * H], w1_w2f[H:2 * H]])
        b_fo, a_w2f = _multi_matmul(
            xf, [w1_fo[H:2 * H], w1_w2f[:H]])
        aggr_w2o = _edge_phase(p["w2o"], a_w2o, b_w2o, w2o_idx, e_w2o,
                               e_w2o_n, n_obj)
        aggr_oo = _edge_phase(p["oo"], a_oo, b_oo, oo_idx, e_oo,
                              e_oo_n, n_obj)
        aggr_fo = _edge_phase(p["fo"], a_fo, b_fo, fo_idx, e_fo,
                              e_fo_n, n_obj)
        aggr_w2f = _edge_phase(p["w2f"], a_w2f, b_w2f, w2f_idx, e_w2f,
                               e_w2f_n, n_floor)
        xo = _node_update(xo, [aggr_w2o, aggr_oo, aggr_fo],
                          [p["w2o"]["node_mlp"], p["oo"]["node_mlp"],
                           p["fo"]["node_mlp"]])
        xf = _node_update(xf, [aggr_w2f], [p["w2f"]["node_mlp"]])

    out_object = _mlp_full(xo, params["decoder_object"], ln=False, d_out=3)
    dec = params["decoder_contact"]
    w1_dec = dec["lin"][0][0]
    a_dec_o, b_dec = _multi_matmul(xo, [w1_dec[:H], w1_dec[H:2 * H]])
    (a_dec_f,) = _multi_matmul(xf, [w1_dec[:H]])
    out_oo = _contact_dec(a_dec_o, b_dec, oo_idx, e_oo, e_oo_n, dec)
    out_fo = _contact_dec(a_dec_f, b_dec, fo_idx, e_fo, e_fo_n, dec)
    return (out_object, out_oo, out_fo)


# speedup vs baseline: 1.0147x; 1.0147x over previous
import functools

import jax
import jax.numpy as jnp
from jax import lax
from jax.experimental import pallas as pl
from jax.experimental.pallas import tpu as pltpu
from jax.experimental.pallas import tpu_sc as plsc

H = 128
_BR = 512

try:
    _sci = plsc.get_sparse_core_info()
    _NC, _NS = _sci.num_cores, _sci.num_subcores
except Exception:
    _NC, _NS = 2, 16
_NW = _NC * _NS
_CH = 128
_HC = H // _NC


def _cdiv(a, b):
    return (a + b - 1) // b


def _ln(y, g, be):
    mu = jnp.mean(y, axis=-1, keepdims=True)
    var = jnp.mean((y - mu) ** 2, axis=-1, keepdims=True)
    return (y - mu) * lax.rsqrt(var + 1e-5) * g + be


def _dot(a, b):
    return jnp.dot(a, b, preferred_element_type=jnp.float32)


def _mlp_body(x_ref, w1_ref, b1_ref, w2_ref, b2_ref, w3_ref, b3_ref,
              g_ref, be_ref, o_ref, *, ln):
    h = jnp.maximum(_dot(x_ref[...], w1_ref[...]) + b1_ref[...], 0.0)
    h = jnp.maximum(_dot(h, w2_ref[...]) + b2_ref[...], 0.0)
    y = _dot(h, w3_ref[...]) + b3_ref[...]
    if ln:
        y = _ln(y, g_ref[...], be_ref[...])
    o_ref[...] = y


def _full(shape):
    return pl.BlockSpec(shape, lambda i: (0,) * len(shape))


def _rows(br, cols):
    return pl.BlockSpec((br, cols), lambda i: (i, 0))


def _mlp_full(x, p, ln, d_out=H):
    n, d_in = x.shape
    dp = max(8, _cdiv(d_in, 8) * 8)
    if d_in != dp:
        x = jnp.pad(x, ((0, 0), (0, dp - d_in)))
    (w1, b1), (w2, b2), (w3, b3) = p["lin"]
    w1 = jnp.pad(w1, ((0, dp - d_in), (0, 0)))
    if w3.shape[1] != H:
        w3 = jnp.pad(w3, ((0, 0), (0, H - w3.shape[1])))
        b3 = jnp.pad(b3, ((0, H - b3.shape[0]),))
    if p["ln"] is not None:
        g, be = p["ln"]
    else:
        g = be = jnp.zeros((H,), jnp.float32)
    br = min(_BR, _cdiv(n, 8) * 8)
    out = pl.pallas_call(
        functools.partial(_mlp_body, ln=ln),
        grid=(_cdiv(n, br),),
        in_specs=[_rows(br, dp), _full((dp, H)), _full((1, H)),
                  _full((H, H)), _full((1, H)), _full((H, H)), _full((1, H)),
                  _full((1, H)), _full((1, H))],
        out_specs=_rows(br, H),
        out_shape=jax.ShapeDtypeStruct((n, H), jnp.float32),
    )(x, w1, b1.reshape(1, H), w2, b2.reshape(1, H), w3, b3.reshape(1, H),
      g.reshape(1, H), be.reshape(1, H))
    return out[:, :d_out] if d_out != H else out


def _matmul_body(x_ref, w_ref, b_ref, o_ref):
    o_ref[...] = _dot(x_ref[...], w_ref[...]) + b_ref[...]


def _multi_matmul_body(x_ref, w_ref, *o_refs):
    x = x_ref[...]
    for r, o_ref in enumerate(o_refs):
        o_ref[...] = _dot(x, w_ref[r])


def _multi_matmul(x, ws):
    n = x.shape[0]
    k = len(ws)
    br = min(_BR, _cdiv(n, 8) * 8)
    rows = _rows(br, H)
    return pl.pallas_call(
        _multi_matmul_body,
        grid=(_cdiv(n, br),),
        in_specs=[rows, _full((k, H, H))],
        out_specs=(rows,) * k,
        out_shape=(jax.ShapeDtypeStruct((n, H), jnp.float32),) * k,
    )(x, jnp.stack(ws))


def _matmul(x, w, b=None):
    n = x.shape[0]
    if b is None:
        b = jnp.zeros((H,), jnp.float32)
    br = min(_BR, _cdiv(n, 8) * 8)
    return pl.pallas_call(
        _matmul_body,
        grid=(_cdiv(n, br),),
        in_specs=[_rows(br, H), _full((H, H)), _full((1, H))],
        out_specs=_rows(br, H),
        out_shape=jax.ShapeDtypeStruct((n, H), jnp.float32),
    )(x, w, b.reshape(1, H))


def _tail_body(a_ref, b_ref, e_ref, w1e_ref, b1_ref, w2_ref, b2_ref,
               w3_ref, b3_ref, g_ref, be_ref, *o_refs,
               ln, n_valid, br, split):
    c = _dot(e_ref[...], w1e_ref[...]) + b1_ref[...]
    h = jnp.maximum(a_ref[...] + b_ref[...] + c, 0.0)
    h = jnp.maximum(_dot(h, w2_ref[...]) + b2_ref[...], 0.0)
    y = _dot(h, w3_ref[...]) + b3_ref[...]
    if ln:
        y = _ln(y, g_ref[...], be_ref[...])
    if n_valid is not None:
        row = (pl.program_id(0) * br
               + lax.broadcasted_iota(jnp.int32, y.shape, 0))
        y = jnp.where(row < n_valid, y, 0.0)
    if split:
        o_refs[0][...] = y[:, :_HC]
        o_refs[1][...] = y[:, _HC:]
    else:
        o_refs[0][...] = y


def _tail(ag, bg, e_enc, p, ln, d_out=H, n_valid=None, split=False):
    n = ag.shape[0]
    (w1, b1), (w2, b2), (w3, b3) = p["lin"]
    w1e = w1[2 * H:]
    if w3.shape[1] != H:
        w3 = jnp.pad(w3, ((0, 0), (0, H - w3.shape[1])))
        b3 = jnp.pad(b3, ((0, H - b3.shape[0]),))
    if p["ln"] is not None:
        g, be = p["ln"]
    else:
        g = be = jnp.zeros((H,), jnp.float32)
    br = min(_BR, _cdiv(n, 8) * 8)
    if split:
        out_specs = (_rows(br, _HC), _rows(br, _HC))
        out_shape = (jax.ShapeDtypeStruct((n, _HC), jnp.float32),
                     jax.ShapeDtypeStruct((n, _HC), jnp.float32))
    else:
        out_specs = _rows(br, H)
        out_shape = jax.ShapeDtypeStruct((n, H), jnp.float32)
    out = pl.pallas_call(
        functools.partial(_tail_body, ln=ln, n_valid=n_valid, br=br,
                          split=split),
        grid=(_cdiv(n, br),),
        in_specs=[_rows(br, H), _rows(br, H), _rows(br, H),
                  _full((H, H)), _full((1, H)),
                  _full((H, H)), _full((1, H)), _full((H, H)), _full((1, H)),
                  _full((1, H)), _full((1, H))],
        out_specs=out_specs,
        out_shape=out_shape,
    )(ag, bg, e_enc, w1e, b1.reshape(1, H), w2, b2.reshape(1, H),
      w3, b3.reshape(1, H), g.reshape(1, H), be.reshape(1, H))
    if split:
        return out
    return out[:, :d_out] if d_out != H else out


def _node_body(*refs, nrel):
    x_ref = refs[0]
    a_refs = refs[1:1 + 2 * nrel]
    w1x, w1a, b1, w2, b2, w3, b3, g, be = refs[1 + 2 * nrel:10 + 2 * nrel]
    o_ref = refs[10 + 2 * nrel]
    x = x_ref[...]
    acc = x
    for r in range(nrel):
        a = jnp.concatenate([a_refs[2 * r][...], a_refs[2 * r + 1][...]],
                            axis=-1)
        h = jnp.maximum(_dot(x, w1x[r]) + _dot(a, w1a[r]) + b1[r], 0.0)
        h = jnp.maximum(_dot(h, w2[r]) + b2[r], 0.0)
        acc = acc + _ln(_dot(h, w3[r]) + b3[r], g[r], be[r])
    o_ref[...] = acc


def _node_update(x, aggrs, ps):
    n = x.shape[0]
    nrel = len(aggrs)
    w1x = jnp.stack([p["lin"][0][0][:H] for p in ps])
    w1a = jnp.stack([p["lin"][0][0][H:] for p in ps])
    b1 = jnp.stack([p["lin"][0][1].reshape(1, H) for p in ps])
    w2 = jnp.stack([p["lin"][1][0] for p in ps])
    b2 = jnp.stack([p["lin"][1][1].reshape(1, H) for p in ps])
    w3 = jnp.stack([p["lin"][2][0] for p in ps])
    b3 = jnp.stack([p["lin"][2][1].reshape(1, H) for p in ps])
    g = jnp.stack([p["ln"][0].reshape(1, H) for p in ps])
    be = jnp.stack([p["ln"][1].reshape(1, H) for p in ps])
    br = min(_BR, _cdiv(n, 8) * 8)
    rows = _rows(br, H)
    half = _rows(br, _HC)
    flat_aggrs = [h for pair in aggrs for h in pair]
    return pl.pallas_call(
        functools.partial(_node_body, nrel=nrel),
        grid=(_cdiv(n, br),),
        in_specs=[rows] + [half] * (2 * nrel)
        + [_full((nrel, H, H)), _full((nrel, H, H)), _full((nrel, 1, H)),
           _full((nrel, H, H)), _full((nrel, 1, H)),
           _full((nrel, H, H)), _full((nrel, 1, H)),
           _full((nrel, 1, H)), _full((nrel, 1, H))],
        out_specs=rows,
        out_shape=jax.ShapeDtypeStruct((n, H), jnp.float32),
    )(x, *flat_aggrs, w1x, w1a, b1, w2, b2, w3, b3, g, be)


def _pad_edges(e):
    m = _NW * _CH
    return _cdiv(e, m) * m


def _prep_idx(idx, e_pad, groups):
    idx = idx.astype(jnp.int32)
    return jnp.pad(idx, (0, e_pad - idx.shape[0])).reshape(
        groups, e_pad // (groups * _CH), _CH)


@functools.lru_cache(maxsize=None)
def _make_dual_gather(n1, n2, e_pad):
    nck = e_pad // (_NW * _CH)
    mesh = plsc.VectorSubcoreMesh(core_axis_name="c", subcore_axis_name="s")

    @functools.partial(
        pl.kernel, mesh=mesh,
        out_type=(jax.ShapeDtypeStruct((e_pad, H), jnp.float32),
                  jax.ShapeDtypeStruct((e_pad, H), jnp.float32)),
        scratch_types=[pltpu.VMEM((nck, _CH), jnp.int32),
                       pltpu.VMEM((nck, _CH), jnp.int32),
                       pltpu.VMEM((_CH, H), jnp.float32),
                       pltpu.VMEM((_CH, H), jnp.float32),
                       pltpu.SemaphoreType.DMA,
                       pltpu.SemaphoreType.DMA],
    )
    def k(t1, i1, t2, i2, o1, o2, iv1, iv2, b1, b2, s1, s2):
        wid = lax.axis_index("s") * _NC + lax.axis_index("c")
        pltpu.sync_copy(i1.at[wid], iv1)
        pltpu.sync_copy(i2.at[wid], iv2)

        def chunk(j, carry):
            row0 = (wid * nck + j) * _CH
            cp1 = pltpu.async_copy(t1.at[iv1.at[j]], b1, s1)
            cp2 = pltpu.async_copy(t2.at[iv2.at[j]], b2, s2)
            cp1.wait()
            pltpu.sync_copy(b1, o1.at[pl.ds(row0, _CH)])
            cp2.wait()
            pltpu.sync_copy(b2, o2.at[pl.ds(row0, _CH)])
            return carry

        lax.fori_loop(0, nck, chunk, 0)

    return k


@functools.lru_cache(maxsize=None)
def _make_dual_gather_pipe(n1, n2, e_pad):
    nck = e_pad // (_NW * _CH)
    mesh = plsc.VectorSubcoreMesh(core_axis_name="c", subcore_axis_name="s")

    @functools.partial(
        pl.kernel, mesh=mesh,
        out_type=(jax.ShapeDtypeStruct((e_pad, H), jnp.float32),
                  jax.ShapeDtypeStruct((e_pad, H), jnp.float32)),
        scratch_types=[pltpu.VMEM((nck, _CH), jnp.int32),
                       pltpu.VMEM((nck, _CH), jnp.int32),
                       pltpu.VMEM((_CH, H), jnp.float32),
                       pltpu.VMEM((_CH, H), jnp.float32),
                       pltpu.VMEM((_CH, H), jnp.float32),
                       pltpu.VMEM((_CH, H), jnp.float32),
                       pltpu.SemaphoreType.DMA, pltpu.SemaphoreType.DMA,
                       pltpu.SemaphoreType.DMA, pltpu.SemaphoreType.DMA],
    )
    def k(t1, i1, t2, i2, o1, o2, iv1, iv2, a0, a1, b0, b1,
          sa0, sa1, sb0, sb1):
        wid = lax.axis_index("s") * _NC + lax.axis_index("c")
        pltpu.sync_copy(i1.at[wid], iv1)
        pltpu.sync_copy(i2.at[wid], iv2)
        base = wid * nck

        def issue(j, ba, bb, sa, sb):
            pltpu.async_copy(t1.at[iv1.at[j]], ba, sa)
            pltpu.async_copy(t2.at[iv2.at[j]], bb, sb)

        def wait_a(buf, sem):
            pltpu.make_async_copy(t1.at[iv1.at[0]], buf, sem).wait()

        def wait_b(buf, sem):
            pltpu.make_async_copy(t2.at[iv2.at[0]], buf, sem).wait()

        issue(0, a0, b0, sa0, sb0)
        if nck > 1:
            issue(1, a1, b1, sa1, sb1)

        def half(k0, ba, bb, sa, sb):
            wait_a(ba, sa)
            pltpu.sync_copy(ba, o1.at[pl.ds((base + k0) * _CH, _CH)])

            @pl.when(k0 + 2 < nck)
            def _():
                pltpu.async_copy(t1.at[iv1.at[k0 + 2]], ba, sa)

            wait_b(bb, sb)
            pltpu.sync_copy(bb, o2.at[pl.ds((base + k0) * _CH, _CH)])

            @pl.when(k0 + 2 < nck)
            def _():
                pltpu.async_copy(t2.at[iv2.at[k0 + 2]], bb, sb)

        def body(jj, carry):
            half(2 * jj, a0, b0, sa0, sb0)
            half(2 * jj + 1, a1, b1, sa1, sb1)
            return carry

        lax.fori_loop(0, nck // 2, body, 0)
        if nck % 2 == 1:
            j = nck - 1
            wait_a(a0, sa0)
            pltpu.sync_copy(a0, o1.at[pl.ds((base + j) * _CH, _CH)])
            wait_b(b0, sb0)
            pltpu.sync_copy(b0, o2.at[pl.ds((base + j) * _CH, _CH)])

    return k


_PIPELINE = True
_PROBE_MIN = True


def _dual_gather(tab1, idx1_3d, tab2, idx2_3d):
    e_pad = idx1_3d.shape[0] * idx1_3d.shape[1] * _CH
    mk = _make_dual_gather_pipe if _PIPELINE else _make_dual_gather
    return mk(tab1.shape[0], tab2.shape[0], e_pad)(
        tab1, idx1_3d, tab2, idx2_3d)


@functools.lru_cache(maxsize=None)
def _make_scatter(e_pad, n_pad):
    nck = e_pad // (_NS * _CH)
    rows_pt = n_pad // _NS
    mesh = plsc.VectorSubcoreMesh(core_axis_name="c", subcore_axis_name="s")

    @functools.partial(
        pl.kernel, mesh=mesh,
        out_type=(jax.ShapeDtypeStruct((n_pad, _HC), jnp.float32),
                  jax.ShapeDtypeStruct((n_pad, _HC), jnp.float32)),
        scratch_types=[pltpu.VMEM((nck, _CH), jnp.int32),
                       pltpu.VMEM((_CH, _HC), jnp.float32),
                       pltpu.VMEM_SHARED((n_pad, _HC), jnp.float32)],
    )
    def k(msg_lo, msg_hi, idx, zeros, out_lo, out_hi, iv, mbuf, acc):
        c = lax.axis_index("c")
        s = lax.axis_index("s")
        r0 = s * rows_pt
        if _PROBE_MIN:
            pltpu.sync_copy(zeros.at[pl.ds(0, min(_CH, rows_pt))],
                            mbuf.at[pl.ds(0, min(_CH, rows_pt))])
            for r in range(0, rows_pt, _CH):
                sz = min(_CH, rows_pt - r)
                pltpu.sync_copy(mbuf.at[pl.ds(0, sz)],
                                acc.at[pl.ds(r0 + r, sz)])
            plsc.subcore_barrier()
            for r in range(0, rows_pt, _CH):
                sz = min(_CH, rows_pt - r)
                pltpu.sync_copy(acc.at[pl.ds(r0 + r, sz)],
                                mbuf.at[pl.ds(0, sz)])

                @pl.when(c == 0)
                def _():
                    pltpu.sync_copy(mbuf.at[pl.ds(0, sz)],
                                    out_lo.at[pl.ds(r0 + r, sz)])

                @pl.when(c == 1)
                def _():
                    pltpu.sync_copy(mbuf.at[pl.ds(0, sz)],
                                    out_hi.at[pl.ds(r0 + r, sz)])
            return
        pltpu.sync_copy(zeros.at[pl.ds(0, min(_CH, rows_pt))],
                        mbuf.at[pl.ds(0, min(_CH, rows_pt))])
        for r in range(0, rows_pt, _CH):
            sz = min(_CH, rows_pt - r)
            pltpu.sync_copy(mbuf.at[pl.ds(0, sz)],
                            acc.at[pl.ds(r0 + r, sz)])
        pltpu.sync_copy(idx.at[s], iv)
        plsc.subcore_barrier()

        def chunk(j, carry):
            row0 = (s * nck + j) * _CH
            rows = pl.ds(row0, _CH)

            @pl.when(c == 0)
            def _():
                pltpu.sync_copy(msg_lo.at[rows], mbuf)

            @pl.when(c == 1)
            def _():
                pltpu.sync_copy(msg_hi.at[rows], mbuf)

            pltpu.sync_copy(mbuf, acc.at[iv.at[j]], add=True)
            return carry

        lax.fori_loop(0, nck, chunk, 0)
        plsc.subcore_barrier()
        for r in range(0, rows_pt, _CH):
            sz = min(_CH, rows_pt - r)
            pltpu.sync_copy(acc.at[pl.ds(r0 + r, sz)],
                            mbuf.at[pl.ds(0, sz)])

            @pl.when(c == 0)
            def _():
                pltpu.sync_copy(mbuf.at[pl.ds(0, sz)],
                                out_lo.at[pl.ds(r0 + r, sz)])

            @pl.when(c == 1)
            def _():
                pltpu.sync_copy(mbuf.at[pl.ds(0, sz)],
                                out_hi.at[pl.ds(r0 + r, sz)])

    return k


@functools.lru_cache(maxsize=None)
def _make_scatter_pipe(e_pad, n_pad):
    nck = e_pad // (_NS * _CH)
    rows_pt = n_pad // _NS
    mesh = plsc.VectorSubcoreMesh(core_axis_name="c", subcore_axis_name="s")

    @functools.partial(
        pl.kernel, mesh=mesh,
        out_type=(jax.ShapeDtypeStruct((n_pad, _HC), jnp.float32),
                  jax.ShapeDtypeStruct((n_pad, _HC), jnp.float32)),
        scratch_types=[pltpu.VMEM((nck, _CH), jnp.int32),
                       pltpu.VMEM((_CH, _HC), jnp.float32),
                       pltpu.VMEM((_CH, _HC), jnp.float32),
                       pltpu.VMEM_SHARED((n_pad, _HC), jnp.float32),
                       pltpu.SemaphoreType.DMA, pltpu.SemaphoreType.DMA],
    )
    def k(msg_lo, msg_hi, idx, zeros, out_lo, out_hi, iv, m0, m1, acc,
          sm0, sm1):
        c = lax.axis_index("c")
        s = lax.axis_index("s")
        r0 = s * rows_pt
        pltpu.sync_copy(zeros.at[pl.ds(r0, rows_pt)],
                        acc.at[pl.ds(r0, rows_pt)])
        pltpu.sync_copy(idx.at[s], iv)
        plsc.subcore_barrier()

        def load(j, buf, sem):
            rows = pl.ds((s * nck + j) * _CH, _CH)

            @pl.when(c == 0)
            def _():
                pltpu.async_copy(msg_lo.at[rows], buf, sem)

            @pl.when(c == 1)
            def _():
                pltpu.async_copy(msg_hi.at[rows], buf, sem)

        def wait_m(buf, sem):
            pltpu.make_async_copy(msg_lo.at[pl.ds(0, _CH)], buf, sem).wait()

        load(0, m0, sm0)
        load(1, m1, sm1)

        def half(k0, buf, sem):
            wait_m(buf, sem)
            pltpu.sync_copy(buf, acc.at[iv.at[pl.ds(k0, 1)]], add=True)

            @pl.when(k0 + 2 < nck)
            def _():
                load_j = k0 + 2
                rows = pl.ds((s * nck + load_j) * _CH, _CH)

                @pl.when(c == 0)
                def _():
                    pltpu.async_copy(msg_lo.at[rows], buf, sem)

                @pl.when(c == 1)
                def _():
                    pltpu.async_copy(msg_hi.at[rows], buf, sem)

        def body(jj, carry):
            half(2 * jj, m0, sm0)
            half(2 * jj + 1, m1, sm1)
            return carry

        lax.fori_loop(0, nck // 2, body, 0)
        plsc.subcore_barrier()
        rows = pl.ds(r0, rows_pt)

        @pl.when(c == 0)
        def _():
            pltpu.sync_copy(acc.at[rows], out_lo.at[rows])

        @pl.when(c == 1)
        def _():
            pltpu.sync_copy(acc.at[rows], out_hi.at[rows])

    return k


def _segment_sum(msg_halves, dst_3d, n):
    msg_lo, msg_hi = msg_halves
    n_pad = _cdiv(n, _NS * 8) * (_NS * 8)
    dst = dst_3d.reshape(-1)
    lo = jax.ops.segment_sum(msg_lo, dst, num_segments=n_pad)
    hi = jax.ops.segment_sum(msg_hi, dst, num_segments=n_pad)
    return lo, hi


def _edge_phase(p_in, a, b, idxs, e_enc, e_true, n_dst):
    src_g, dst_g, dst_s = idxs
    em = p_in["edge_mlp"]
    ag, bg = _dual_gather(a, dst_g, b, src_g)
    msg = _tail(ag, bg, e_enc, em, ln=True, n_valid=e_true, split=True)
    return _segment_sum(msg, dst_s, n_dst)


def _contact_dec(a, b, idxs, e_enc, e_true, p):
    src_g, dst_g, _ = idxs
    ag, bg = _dual_gather(a, src_g, b, dst_g)
    return _tail(ag, bg, e_enc, p, ln=False, d_out=1, n_valid=e_true)[:e_true]


def kernel(world_x, object_x, floor_x, w2o_edge_index, w2o_edge_attr,
           w2f_edge_index, w2f_edge_attr, oo_edge_index, oo_edge_attr,
           fo_edge_index, fo_edge_attr, params):
    e_w2o_n, e_w2f_n = w2o_edge_attr.shape[0], w2f_edge_attr.shape[0]
    e_oo_n, e_fo_n = oo_edge_attr.shape[0], fo_edge_attr.shape[0]
    p_w2o, p_w2f = _pad_edges(e_w2o_n), _pad_edges(e_w2f_n)
    p_oo, p_fo = _pad_edges(e_oo_n), _pad_edges(e_fo_n)

    def pad_attr(x, e_pad):
        return jnp.pad(x, ((0, e_pad - x.shape[0]), (0, 0)))

    def prep(ei, e_pad):
        return (_prep_idx(ei[0], e_pad, _NW),
                _prep_idx(ei[1], e_pad, _NW),
                _prep_idx(ei[1], e_pad, _NS))

    w2o_idx = prep(w2o_edge_index, p_w2o)
    w2f_idx = prep(w2f_edge_index, p_w2f)
    oo_idx = prep(oo_edge_index, p_oo)
    fo_idx = prep(fo_edge_index, p_fo)

    xw = _mlp_full(world_x, params["node_enc_world"], ln=True)
    xo = _mlp_full(object_x, params["node_enc_object"], ln=True)
    xf = _mlp_full(floor_x, params["node_enc_floor"], ln=True)
    e_w2o = _mlp_full(pad_attr(w2o_edge_attr, p_w2o),
                      params["edge_enc_w2o"], ln=True)
    e_w2f = _mlp_full(pad_attr(w2f_edge_attr, p_w2f),
                      params["edge_enc_w2f"], ln=True)
    e_oo = _mlp_full(pad_attr(oo_edge_attr, p_oo),
                     params["edge_enc_contact"], ln=True)
    e_fo = _mlp_full(pad_attr(fo_edge_attr, p_fo),
                     params["edge_enc_contact"], ln=True)

    n_obj = xo.shape[0]
    n_floor = xf.shape[0]
    for p in params["processor"]:
        w1_w2o = p["w2o"]["edge_mlp"]["lin"][0][0]
        w1_oo = p["oo"]["edge_mlp"]["lin"][0][0]
        w1_fo = p["fo"]["edge_mlp"]["lin"][0][0]
        w1_w2f = p["w2f"]["edge_mlp"]["lin"][0][0]
        a_w2o, a_oo, b_oo, a_fo = _multi_matmul(
            xo, [w1_w2o[:H], w1_oo[:H], w1_oo[H:2 * H], w1_fo[:H]])
        b_w2o, b_w2f = _multi_matmul(
            xw, [w1_w2o[H:2 * H], w1_w2f[H:2 * H]])
        b_fo, a_w2f = _multi_matmul(
            xf, [w1_fo[H:2 * H], w1_w2f[:H]])
        aggr_w2o = _edge_phase(p["w2o"], a_w2o, b_w2o, w2o_idx, e_w2o,
                               e_w2o_n, n_obj)
        aggr_oo = _edge_phase(p["oo"], a_oo, b_oo, oo_idx, e_oo,
                              e_oo_n, n_obj)
        aggr_fo = _edge_phase(p["fo"], a_fo, b_fo, fo_idx, e_fo,
                              e_fo_n, n_obj)
        aggr_w2f = _edge_phase(p["w2f"], a_w2f, b_w2f, w2f_idx, e_w2f,
                               e_w2f_n, n_floor)
        xo = _node_update(xo, [aggr_w2o, aggr_oo, aggr_fo],
                          [p["w2o"]["node_mlp"], p["oo"]["node_mlp"],
                           p["fo"]["node_mlp"]])
        xf = _node_update(xf, [aggr_w2f], [p["w2f"]["node_mlp"]])

    out_object = _mlp_full(xo, params["decoder_object"], ln=False, d_out=3)
    dec = params["decoder_contact"]
    w1_dec = dec["lin"][0][0]
    a_dec_o, b_dec = _multi_matmul(xo, [w1_dec[:H], w1_dec[H:2 * H]])
    (a_dec_f,) = _multi_matmul(xf, [w1_dec[:H]])
    out_oo = _contact_dec(a_dec_o, b_dec, oo_idx, e_oo, e_oo_n, dec)
    out_fo = _contact_dec(a_dec_f, b_dec, fo_idx, e_fo, e_fo_n, dec)
    return (out_object, out_oo, out_fo)


# speedup vs baseline: 1.0150x; 1.0002x over previous
import functools

import jax
import jax.numpy as jnp
from jax import lax
from jax.experimental import pallas as pl
from jax.experimental.pallas import tpu as pltpu
from jax.experimental.pallas import tpu_sc as plsc

H = 128
_BR = 512

try:
    _sci = plsc.get_sparse_core_info()
    _NC, _NS = _sci.num_cores, _sci.num_subcores
except Exception:
    _NC, _NS = 2, 16
_NW = _NC * _NS
_CH = 128
_HC = H // _NC


def _cdiv(a, b):
    return (a + b - 1) // b


def _ln(y, g, be):
    mu = jnp.mean(y, axis=-1, keepdims=True)
    var = jnp.mean((y - mu) ** 2, axis=-1, keepdims=True)
    return (y - mu) * lax.rsqrt(var + 1e-5) * g + be


def _dot(a, b):
    return jnp.dot(a, b, preferred_element_type=jnp.float32)


def _mlp_body(x_ref, w1_ref, b1_ref, w2_ref, b2_ref, w3_ref, b3_ref,
              g_ref, be_ref, o_ref, *, ln):
    h = jnp.maximum(_dot(x_ref[...], w1_ref[...]) + b1_ref[...], 0.0)
    h = jnp.maximum(_dot(h, w2_ref[...]) + b2_ref[...], 0.0)
    y = _dot(h, w3_ref[...]) + b3_ref[...]
    if ln:
        y = _ln(y, g_ref[...], be_ref[...])
    o_ref[...] = y


def _full(shape):
    return pl.BlockSpec(shape, lambda i: (0,) * len(shape))


def _rows(br, cols):
    return pl.BlockSpec((br, cols), lambda i: (i, 0))


def _mlp_full(x, p, ln, d_out=H):
    n, d_in = x.shape
    dp = max(8, _cdiv(d_in, 8) * 8)
    if d_in != dp:
        x = jnp.pad(x, ((0, 0), (0, dp - d_in)))
    (w1, b1), (w2, b2), (w3, b3) = p["lin"]
    w1 = jnp.pad(w1, ((0, dp - d_in), (0, 0)))
    if w3.shape[1] != H:
        w3 = jnp.pad(w3, ((0, 0), (0, H - w3.shape[1])))
        b3 = jnp.pad(b3, ((0, H - b3.shape[0]),))
    if p["ln"] is not None:
        g, be = p["ln"]
    else:
        g = be = jnp.zeros((H,), jnp.float32)
    br = min(_BR, _cdiv(n, 8) * 8)
    out = pl.pallas_call(
        functools.partial(_mlp_body, ln=ln),
        grid=(_cdiv(n, br),),
        in_specs=[_rows(br, dp), _full((dp, H)), _full((1, H)),
                  _full((H, H)), _full((1, H)), _full((H, H)), _full((1, H)),
                  _full((1, H)), _full((1, H))],
        out_specs=_rows(br, H),
        out_shape=jax.ShapeDtypeStruct((n, H), jnp.float32),
    )(x, w1, b1.reshape(1, H), w2, b2.reshape(1, H), w3, b3.reshape(1, H),
      g.reshape(1, H), be.reshape(1, H))
    return out[:, :d_out] if d_out != H else out


def _multi_matmul_body(x_ref, w_ref, *o_refs):
    x = x_ref[...]
    for r, o_ref in enumerate(o_refs):
        o_ref[...] = _dot(x, w_ref[r])


def _multi_matmul(x, ws):
    n = x.shape[0]
    k = len(ws)
    br = min(_BR, _cdiv(n, 8) * 8)
    rows = _rows(br, H)
    return pl.pallas_call(
        _multi_matmul_body,
        grid=(_cdiv(n, br),),
        in_specs=[rows, _full((k, H, H))],
        out_specs=(rows,) * k,
        out_shape=(jax.ShapeDtypeStruct((n, H), jnp.float32),) * k,
    )(x, jnp.stack(ws))


def _tail_body(a_ref, b_ref, e_ref, w1e_ref, b1_ref, w2_ref, b2_ref,
               w3_ref, b3_ref, g_ref, be_ref, *o_refs,
               ln, n_valid, br, split):
    c = _dot(e_ref[...], w1e_ref[...]) + b1_ref[...]
    h = jnp.maximum(a_ref[...] + b_ref[...] + c, 0.0)
    h = jnp.maximum(_dot(h, w2_ref[...]) + b2_ref[...], 0.0)
    y = _dot(h, w3_ref[...]) + b3_ref[...]
    if ln:
        y = _ln(y, g_ref[...], be_ref[...])
    if n_valid is not None:
        row = (pl.program_id(0) * br
               + lax.broadcasted_iota(jnp.int32, y.shape, 0))
        y = jnp.where(row < n_valid, y, 0.0)
    if split:
        o_refs[0][...] = y[:, :_HC]
        o_refs[1][...] = y[:, _HC:]
    else:
        o_refs[0][...] = y


def _tail(ag, bg, e_enc, p, ln, d_out=H, n_valid=None, split=False):
    n = ag.shape[0]
    (w1, b1), (w2, b2), (w3, b3) = p["lin"]
    w1e = w1[2 * H:]
    if w3.shape[1] != H:
        w3 = jnp.pad(w3, ((0, 0), (0, H - w3.shape[1])))
        b3 = jnp.pad(b3, ((0, H - b3.shape[0]),))
    if p["ln"] is not None:
        g, be = p["ln"]
    else:
        g = be = jnp.zeros((H,), jnp.float32)
    br = min(_BR, _cdiv(n, 8) * 8)
    if split:
        out_specs = (_rows(br, _HC), _rows(br, _HC))
        out_shape = (jax.ShapeDtypeStruct((n, _HC), jnp.float32),
                     jax.ShapeDtypeStruct((n, _HC), jnp.float32))
    else:
        out_specs = _rows(br, H)
        out_shape = jax.ShapeDtypeStruct((n, H), jnp.float32)
    out = pl.pallas_call(
        functools.partial(_tail_body, ln=ln, n_valid=n_valid, br=br,
                          split=split),
        grid=(_cdiv(n, br),),
        in_specs=[_rows(br, H), _rows(br, H), _rows(br, H),
                  _full((H, H)), _full((1, H)),
                  _full((H, H)), _full((1, H)), _full((H, H)), _full((1, H)),
                  _full((1, H)), _full((1, H))],
        out_specs=out_specs,
        out_shape=out_shape,
    )(ag, bg, e_enc, w1e, b1.reshape(1, H), w2, b2.reshape(1, H),
      w3, b3.reshape(1, H), g.reshape(1, H), be.reshape(1, H))
    if split:
        return out
    return out[:, :d_out] if d_out != H else out


def _node_body(*refs, nrel):
    x_ref = refs[0]
    a_refs = refs[1:1 + 2 * nrel]
    w1x, w1a, b1, w2, b2, w3, b3, g, be = refs[1 + 2 * nrel:10 + 2 * nrel]
    o_ref = refs[10 + 2 * nrel]
    x = x_ref[...]
    acc = x
    for r in range(nrel):
        a = jnp.concatenate([a_refs[2 * r][...], a_refs[2 * r + 1][...]],
                            axis=-1)
        h = jnp.maximum(_dot(x, w1x[r]) + _dot(a, w1a[r]) + b1[r], 0.0)
        h = jnp.maximum(_dot(h, w2[r]) + b2[r], 0.0)
        acc = acc + _ln(_dot(h, w3[r]) + b3[r], g[r], be[r])
    o_ref[...] = acc


def _node_update(x, aggrs, ps):
    n = x.shape[0]
    nrel = len(aggrs)
    w1x = jnp.stack([p["lin"][0][0][:H] for p in ps])
    w1a = jnp.stack([p["lin"][0][0][H:] for p in ps])
    b1 = jnp.stack([p["lin"][0][1].reshape(1, H) for p in ps])
    w2 = jnp.stack([p["lin"][1][0] for p in ps])
    b2 = jnp.stack([p["lin"][1][1].reshape(1, H) for p in ps])
    w3 = jnp.stack([p["lin"][2][0] for p in ps])
    b3 = jnp.stack([p["lin"][2][1].reshape(1, H) for p in ps])
    g = jnp.stack([p["ln"][0].reshape(1, H) for p in ps])
    be = jnp.stack([p["ln"][1].reshape(1, H) for p in ps])
    br = min(_BR, _cdiv(n, 8) * 8)
    rows = _rows(br, H)
    half = _rows(br, _HC)
    flat_aggrs = [h for pair in aggrs for h in pair]
    return pl.pallas_call(
        functools.partial(_node_body, nrel=nrel),
        grid=(_cdiv(n, br),),
        in_specs=[rows] + [half] * (2 * nrel)
        + [_full((nrel, H, H)), _full((nrel, H, H)), _full((nrel, 1, H)),
           _full((nrel, H, H)), _full((nrel, 1, H)),
           _full((nrel, H, H)), _full((nrel, 1, H)),
           _full((nrel, 1, H)), _full((nrel, 1, H))],
        out_specs=rows,
        out_shape=jax.ShapeDtypeStruct((n, H), jnp.float32),
    )(x, *flat_aggrs, w1x, w1a, b1, w2, b2, w3, b3, g, be)


def _pad_edges(e):
    m = _NW * _CH
    return _cdiv(e, m) * m


def _prep_idx(idx, e_pad, groups):
    idx = idx.astype(jnp.int32)
    return jnp.pad(idx, (0, e_pad - idx.shape[0])).reshape(
        groups, e_pad // (groups * _CH), _CH)


@functools.lru_cache(maxsize=None)
def _make_dual_gather_pipe(n1, n2, e_pad):
    nck = e_pad // (_NW * _CH)
    mesh = plsc.VectorSubcoreMesh(core_axis_name="c", subcore_axis_name="s")

    @functools.partial(
        pl.kernel, mesh=mesh,
        out_type=(jax.ShapeDtypeStruct((e_pad, H), jnp.float32),
                  jax.ShapeDtypeStruct((e_pad, H), jnp.float32)),
        scratch_types=[pltpu.VMEM((nck, _CH), jnp.int32),
                       pltpu.VMEM((nck, _CH), jnp.int32),
                       pltpu.VMEM((_CH, H), jnp.float32),
                       pltpu.VMEM((_CH, H), jnp.float32),
                       pltpu.VMEM((_CH, H), jnp.float32),
                       pltpu.VMEM((_CH, H), jnp.float32),
                       pltpu.SemaphoreType.DMA, pltpu.SemaphoreType.DMA,
                       pltpu.SemaphoreType.DMA, pltpu.SemaphoreType.DMA],
    )
    def k(t1, i1, t2, i2, o1, o2, iv1, iv2, a0, a1, b0, b1,
          sa0, sa1, sb0, sb1):
        wid = lax.axis_index("s") * _NC + lax.axis_index("c")
        pltpu.sync_copy(i1.at[wid], iv1)
        pltpu.sync_copy(i2.at[wid], iv2)
        base = wid * nck

        def issue(j, ba, bb, sa, sb):
            pltpu.async_copy(t1.at[iv1.at[j]], ba, sa)
            pltpu.async_copy(t2.at[iv2.at[j]], bb, sb)

        def wait_a(buf, sem):
            pltpu.make_async_copy(t1.at[iv1.at[0]], buf, sem).wait()

        def wait_b(buf, sem):
            pltpu.make_async_copy(t2.at[iv2.at[0]], buf, sem).wait()

        issue(0, a0, b0, sa0, sb0)
        if nck > 1:
            issue(1, a1, b1, sa1, sb1)

        def half(k0, ba, bb, sa, sb):
            wait_a(ba, sa)
            pltpu.sync_copy(ba, o1.at[pl.ds((base + k0) * _CH, _CH)])

            @pl.when(k0 + 2 < nck)
            def _():
                pltpu.async_copy(t1.at[iv1.at[k0 + 2]], ba, sa)

            wait_b(bb, sb)
            pltpu.sync_copy(bb, o2.at[pl.ds((base + k0) * _CH, _CH)])

            @pl.when(k0 + 2 < nck)
            def _():
                pltpu.async_copy(t2.at[iv2.at[k0 + 2]], bb, sb)

        def body(jj, carry):
            half(2 * jj, a0, b0, sa0, sb0)
            half(2 * jj + 1, a1, b1, sa1, sb1)
            return carry

        lax.fori_loop(0, nck // 2, body, 0)
        if nck % 2 == 1:
            j = nck - 1
            wait_a(a0, sa0)
            pltpu.sync_copy(a0, o1.at[pl.ds((base + j) * _CH, _CH)])
            wait_b(b0, sb0)
            pltpu.sync_copy(b0, o2.at[pl.ds((base + j) * _CH, _CH)])

    return k


def _dual_gather(tab1, idx1_3d, tab2, idx2_3d):
    e_pad = idx1_3d.shape[0] * idx1_3d.shape[1] * _CH
    return _make_dual_gather_pipe(tab1.shape[0], tab2.shape[0], e_pad)(
        tab1, idx1_3d, tab2, idx2_3d)


def _segment_sum(msg_halves, dst_3d, n):
    msg_lo, msg_hi = msg_halves
    n_pad = _cdiv(n, _NS * 8) * (_NS * 8)
    dst = dst_3d.reshape(-1)
    lo = jax.ops.segment_sum(msg_lo, dst, num_segments=n_pad)
    hi = jax.ops.segment_sum(msg_hi, dst, num_segments=n_pad)
    return lo, hi


def _edge_phase(p_in, a, b, idxs, e_enc, e_true, n_dst):
    src_g, dst_g, dst_s = idxs
    em = p_in["edge_mlp"]
    ag, bg = _dual_gather(a, dst_g, b, src_g)
    msg = _tail(ag, bg, e_enc, em, ln=True, n_valid=e_true, split=True)
    return _segment_sum(msg, dst_s, n_dst)


def _contact_dec(a, b, idxs, e_enc, e_true, p):
    src_g, dst_g, _ = idxs
    ag, bg = _dual_gather(a, src_g, b, dst_g)
    return _tail(ag, bg, e_enc, p, ln=False, d_out=1, n_valid=e_true)[:e_true]


def kernel(world_x, object_x, floor_x, w2o_edge_index, w2o_edge_attr,
           w2f_edge_index, w2f_edge_attr, oo_edge_index, oo_edge_attr,
           fo_edge_index, fo_edge_attr, params):
    e_w2o_n, e_w2f_n = w2o_edge_attr.shape[0], w2f_edge_attr.shape[0]
    e_oo_n, e_fo_n = oo_edge_attr.shape[0], fo_edge_attr.shape[0]
    p_w2o, p_w2f = _pad_edges(e_w2o_n), _pad_edges(e_w2f_n)
    p_oo, p_fo = _pad_edges(e_oo_n), _pad_edges(e_fo_n)

    def pad_attr(x, e_pad):
        return jnp.pad(x, ((0, e_pad - x.shape[0]), (0, 0)))

    def prep(ei, e_pad):
        return (_prep_idx(ei[0], e_pad, _NW),
                _prep_idx(ei[1], e_pad, _NW),
                _prep_idx(ei[1], e_pad, _NS))

    w2o_idx = prep(w2o_edge_index, p_w2o)
    w2f_idx = prep(w2f_edge_index, p_w2f)
    oo_idx = prep(oo_edge_index, p_oo)
    fo_idx = prep(fo_edge_index, p_fo)

    xw = _mlp_full(world_x, params["node_enc_world"], ln=True)
    xo = _mlp_full(object_x, params["node_enc_object"], ln=True)
    xf = _mlp_full(floor_x, params["node_enc_floor"], ln=True)
    e_w2o = _mlp_full(pad_attr(w2o_edge_attr, p_w2o),
                      params["edge_enc_w2o"], ln=True)
    e_w2f = _mlp_full(pad_attr(w2f_edge_attr, p_w2f),
                      params["edge_enc_w2f"], ln=True)
    e_oo = _mlp_full(pad_attr(oo_edge_attr, p_oo),
                     params["edge_enc_contact"], ln=True)
    e_fo = _mlp_full(pad_attr(fo_edge_attr, p_fo),
                     params["edge_enc_contact"], ln=True)

    n_obj = xo.shape[0]
    n_floor = xf.shape[0]
    for p in params["processor"]:
        w1_w2o = p["w2o"]["edge_mlp"]["lin"][0][0]
        w1_oo = p["oo"]["edge_mlp"]["lin"][0][0]
        w1_fo = p["fo"]["edge_mlp"]["lin"][0][0]
        w1_w2f = p["w2f"]["edge_mlp"]["lin"][0][0]
        a_w2o, a_oo, b_oo, a_fo = _multi_matmul(
            xo, [w1_w2o[:H], w1_oo[:H], w1_oo[H:2 * H], w1_fo[:H]])
        b_w2o, b_w2f = _multi_matmul(
            xw, [w1_w2o[H:2 * H], w1_w2f[H:2 * H]])
        b_fo, a_w2f = _multi_matmul(
            xf, [w1_fo[H:2 * H], w1_w2f[:H]])
        aggr_w2o = _edge_phase(p["w2o"], a_w2o, b_w2o, w2o_idx, e_w2o,
                               e_w2o_n, n_obj)
        aggr_oo = _edge_phase(p["oo"], a_oo, b_oo, oo_idx, e_oo,
                              e_oo_n, n_obj)
        aggr_fo = _edge_phase(p["fo"], a_fo, b_fo, fo_idx, e_fo,
                              e_fo_n, n_obj)
        aggr_w2f = _edge_phase(p["w2f"], a_w2f, b_w2f, w2f_idx, e_w2f,
                               e_w2f_n, n_floor)
        xo = _node_update(xo, [aggr_w2o, aggr_oo, aggr_fo],
                          [p["w2o"]["node_mlp"], p["oo"]["node_mlp"],
                           p["fo"]["node_mlp"]])
        xf = _node_update(xf, [aggr_w2f], [p["w2f"]["node_mlp"]])

    out_object = _mlp_full(xo, params["decoder_object"], ln=False, d_out=3)
    dec = params["decoder_contact"]
    w1_dec = dec["lin"][0][0]
    a_dec_o, b_dec = _multi_matmul(xo, [w1_dec[:H], w1_dec[H:2 * H]])
    (a_dec_f,) = _multi_matmul(xf, [w1_dec[:H]])
    out_oo = _contact_dec(a_dec_o, b_dec, oo_idx, e_oo, e_oo_n, dec)
    out_fo = _contact_dec(a_dec_f, b_dec, fo_idx, e_fo, e_fo_n, dec)
    return (out_object, out_oo, out_fo)
